# Initial kernel scaffold; baseline (speedup 1.0000x reference)
import jax
import jax.numpy as jnp
from jax.experimental import pallas as pl


def kernel(x0, edge_index0, x1, edge_index1, W1_0, b1_0, W2_0, b2_0, W1_1, b1_1, W2_1, b2_1):
    raise NotImplementedError("write your pallas kernel here")


# speedup vs baseline: 23.1945x; 23.1945x over previous
import functools

import jax
import jax.numpy as jnp
from jax import lax
from jax.experimental import pallas as pl
from jax.experimental.pallas import tpu as pltpu
from jax.experimental.pallas import tpu_sc as plsc

NN = 10000
EE = 320000
DIN = 128
DHID = 64
DOUT = 128

NTILES = 16
CHUNK = 128
CPT = 157
EPAD = NTILES * CPT * CHUNK
ACC_ROWS = 10240

_mesh = plsc.VectorSubcoreMesh(core_axis_name="c", subcore_axis_name="s")
_sc_params = pltpu.CompilerParams(use_tc_tiling_on_sc=False)


DEGW = 8


@functools.partial(
    pl.kernel,
    mesh=_mesh,
    compiler_params=_sc_params,
    out_type=jax.ShapeDtypeStruct((2 * NN, DEGW), jnp.float32),
    scratch_types=[
        pltpu.VMEM((CPT, CHUNK), jnp.int32),
        pltpu.VMEM((CHUNK, DEGW), jnp.float32),
        pltpu.VMEM_SHARED((ACC_ROWS, DEGW), jnp.float32),
    ],
)
def _deg_kernel(dst_hbm, ones_hbm, deg_hbm, dst_v, ones_v, acc):
    c = lax.axis_index("c")
    s = lax.axis_index("s")
    pltpu.sync_copy(ones_hbm, acc.at[pl.ds(s * 640, 640)])
    pltpu.sync_copy(dst_hbm.at[c, s], dst_v)
    pltpu.sync_copy(ones_hbm.at[pl.ds(0, CHUNK)], ones_v)
    plsc.subcore_barrier()

    def body(j, carry):
        pltpu.sync_copy(ones_v, acc.at[dst_v.at[j]], add=True)
        return carry

    lax.fori_loop(0, CPT, body, 0)
    plsc.subcore_barrier()

    @pl.when(s < NTILES - 1)
    def _():
        pltpu.sync_copy(acc.at[pl.ds(s * 640, 640)],
                        deg_hbm.at[pl.ds(c * NN + s * 640, 640)])

    @pl.when(s == NTILES - 1)
    def _():
        pltpu.sync_copy(acc.at[pl.ds(9600, 400)],
                        deg_hbm.at[pl.ds(c * NN + 9600, 400)])


@functools.partial(
    pl.kernel,
    mesh=_mesh,
    compiler_params=_sc_params,
    out_type=jax.ShapeDtypeStruct((2 * NN, DHID), jnp.float32),
    scratch_types=[
        pltpu.VMEM((CPT, CHUNK), jnp.int32),
        pltpu.VMEM((CPT, CHUNK), jnp.int32),
        pltpu.VMEM((CHUNK, DHID), jnp.float32),
        pltpu.VMEM_SHARED((ACC_ROWS, DHID), jnp.float32),
        pltpu.SemaphoreType.DMA,
    ],
)
def _agg_kernel(vt_hbm, src_hbm, dst_hbm, zrows_hbm, out_hbm,
                src_v, dst_v, gb, acc, sem):
    c = lax.axis_index("c")
    s = lax.axis_index("s")
    pltpu.sync_copy(zrows_hbm, acc.at[pl.ds(s * 640, 640)])
    pltpu.sync_copy(src_hbm.at[c, s], src_v)
    pltpu.sync_copy(dst_hbm.at[c, s], dst_v)
    plsc.subcore_barrier()

    def body(j, carry):
        pltpu.async_copy(vt_hbm.at[src_v.at[j]], gb, sem).wait()
        pltpu.sync_copy(gb, acc.at[dst_v.at[j]], add=True)
        return carry

    lax.fori_loop(0, CPT, body, 0)
    plsc.subcore_barrier()

    @pl.when(s < NTILES - 1)
    def _():
        pltpu.sync_copy(acc.at[pl.ds(s * 640, 640)],
                        out_hbm.at[pl.ds(c * NN + s * 640, 640)])

    @pl.when(s == NTILES - 1)
    def _():
        pltpu.sync_copy(acc.at[pl.ds(9600, 400)],
                        out_hbm.at[pl.ds(c * NN + 9600, 400)])


_BA = 2000


def _stage_a_body(x_ref, w_ref, deg_ref, o_ref):
    dinv = lax.rsqrt(deg_ref[...])
    xw = jnp.dot(x_ref[...], w_ref[...], preferred_element_type=jnp.float32)
    o_ref[...] = xw * dinv


def _stage_b_body(s_ref, v_ref, deg_ref, b_ref, o_ref):
    dinv = lax.rsqrt(deg_ref[...])
    z = dinv * (s_ref[...] + v_ref[...]) + b_ref[...]
    h = jnp.where(z > 0, z, jnp.exp(z) - 1.0)
    o_ref[...] = dinv * h


def _stage_c_body(s_ref, v_ref, deg_ref, w_ref, b_ref, o_ref):
    dinv = lax.rsqrt(deg_ref[...])
    agg = dinv * (s_ref[...] + v_ref[...])
    z = jnp.dot(agg, w_ref[...], preferred_element_type=jnp.float32) + b_ref[...]
    o_ref[...] = jnp.where(z > 0, z, jnp.exp(z) - 1.0)


def _stage_a(xs, ws, degs):
    return pl.pallas_call(
        _stage_a_body,
        grid=(2, NN // _BA),
        in_specs=[
            pl.BlockSpec((None, _BA, DIN), lambda g, i: (g, i, 0)),
            pl.BlockSpec((None, DIN, DHID), lambda g, i: (g, 0, 0)),
            pl.BlockSpec((None, _BA, 1), lambda g, i: (g, i, 0)),
        ],
        out_specs=pl.BlockSpec((None, _BA, DHID), lambda g, i: (g, i, 0)),
        out_shape=jax.ShapeDtypeStruct((2, NN, DHID), jnp.float32),
    )(xs, ws, degs)


def _stage_b(s1, vt1, degs, bs):
    return pl.pallas_call(
        _stage_b_body,
        grid=(2, NN // _BA),
        in_specs=[
            pl.BlockSpec((None, _BA, DHID), lambda g, i: (g, i, 0)),
            pl.BlockSpec((None, _BA, DHID), lambda g, i: (g, i, 0)),
            pl.BlockSpec((None, _BA, 1), lambda g, i: (g, i, 0)),
            pl.BlockSpec((None, 1, DHID), lambda g, i: (g, 0, 0)),
        ],
        out_specs=pl.BlockSpec((None, _BA, DHID), lambda g, i: (g, i, 0)),
        out_shape=jax.ShapeDtypeStruct((2, NN, DHID), jnp.float32),
    )(s1, vt1, degs, bs)


def _stage_c(s2, vt2, degs, ws, bs):
    return pl.pallas_call(
        _stage_c_body,
        grid=(2, NN // _BA),
        in_specs=[
            pl.BlockSpec((None, _BA, DHID), lambda g, i: (g, i, 0)),
            pl.BlockSpec((None, _BA, DHID), lambda g, i: (g, i, 0)),
            pl.BlockSpec((None, _BA, 1), lambda g, i: (g, i, 0)),
            pl.BlockSpec((None, DHID, DOUT), lambda g, i: (g, 0, 0)),
            pl.BlockSpec((None, 1, DOUT), lambda g, i: (g, 0, 0)),
        ],
        out_specs=pl.BlockSpec((None, _BA, DOUT), lambda g, i: (g, i, 0)),
        out_shape=jax.ShapeDtypeStruct((2, NN, DOUT), jnp.float32),
    )(s2, vt2, degs, ws, bs)


def kernel(x0, edge_index0, x1, edge_index1,
           W1_0, b1_0, W2_0, b2_0, W1_1, b1_1, W2_1, b2_1):
    xs = jnp.stack([x0, x1])
    w1 = jnp.stack([W1_0, W1_1])
    w2 = jnp.stack([W2_0, W2_1])
    b1 = jnp.stack([b1_0, b1_1])[:, None, :]
    b2 = jnp.stack([b2_0, b2_1])[:, None, :]

    pad = EPAD - EE

    def prep(ei, g):
        src = jnp.concatenate(
            [ei[0] + g * NN, jnp.full((pad,), g * NN, jnp.int32)])
        dst = jnp.concatenate([ei[1], jnp.full((pad,), NN, jnp.int32)])
        return (src.reshape(NTILES, CPT, CHUNK),
                dst.reshape(NTILES, CPT, CHUNK))

    s0, d0 = prep(edge_index0, 0)
    s1e, d1e = prep(edge_index1, 1)
    srcs = jnp.stack([s0, s1e])
    dsts = jnp.stack([d0, d1e])

    ones640 = jnp.ones((640, DEGW), jnp.float32)
    zrows = jnp.zeros((640, DHID), jnp.float32)

    deg = _deg_kernel(dsts, ones640)[:, :1].reshape(2, NN, 1)
    vt1 = _stage_a(xs, w1, deg)
    s1 = _agg_kernel(vt1.reshape(2 * NN, DHID), srcs, dsts, zrows)
    vt2 = _stage_b(s1.reshape(2, NN, DHID), vt1, deg, b1)
    s2 = _agg_kernel(vt2.reshape(2 * NN, DHID), srcs, dsts, zrows)
    out = _stage_c(s2.reshape(2, NN, DHID), vt2, deg, w2, b2)
    return out.reshape(2 * NN, DOUT)


# speedup vs baseline: 23.7879x; 1.0256x over previous
import functools

import jax
import jax.numpy as jnp
from jax import lax
from jax.experimental import pallas as pl
from jax.experimental.pallas import tpu as pltpu
from jax.experimental.pallas import tpu_sc as plsc

NN = 10000
EE = 320000
DIN = 128
DHID = 64
DOUT = 128

NTILES = 16
CHUNK = 128
CPT = 158
CPR = 160
NPAIR = CPT // 2
EPAD = NTILES * CPR * CHUNK
ACC_ROWS = 10240

_mesh = plsc.VectorSubcoreMesh(core_axis_name="c", subcore_axis_name="s")
_sc_params = pltpu.CompilerParams(use_tc_tiling_on_sc=False)


DEGW = 8


@functools.partial(
    pl.kernel,
    mesh=_mesh,
    compiler_params=_sc_params,
    out_type=jax.ShapeDtypeStruct((2 * NN, DEGW), jnp.float32),
    scratch_types=[
        pltpu.VMEM((CPR, CHUNK), jnp.int32),
        pltpu.VMEM((CHUNK, DEGW), jnp.float32),
        pltpu.VMEM_SHARED((ACC_ROWS, DEGW), jnp.float32),
    ],
)
def _deg_kernel(dst_hbm, ones_hbm, deg_hbm, dst_v, ones_v, acc):
    c = lax.axis_index("c")
    s = lax.axis_index("s")
    pltpu.sync_copy(ones_hbm, acc.at[pl.ds(s * 640, 640)])
    pltpu.sync_copy(dst_hbm.at[c, s], dst_v)
    pltpu.sync_copy(ones_hbm.at[pl.ds(0, CHUNK)], ones_v)
    plsc.subcore_barrier()

    def body(j, carry):
        pltpu.sync_copy(ones_v, acc.at[dst_v.at[j]], add=True)
        return carry

    lax.fori_loop(0, CPT, body, 0)
    plsc.subcore_barrier()

    @pl.when(s < NTILES - 1)
    def _():
        pltpu.sync_copy(acc.at[pl.ds(s * 640, 640)],
                        deg_hbm.at[pl.ds(c * NN + s * 640, 640)])

    @pl.when(s == NTILES - 1)
    def _():
        pltpu.sync_copy(acc.at[pl.ds(9600, 400)],
                        deg_hbm.at[pl.ds(c * NN + 9600, 400)])


@functools.partial(
    pl.kernel,
    mesh=_mesh,
    compiler_params=_sc_params,
    out_type=jax.ShapeDtypeStruct((2 * NN, DHID), jnp.float32),
    scratch_types=[
        pltpu.VMEM((CPR, CHUNK), jnp.int32),
        pltpu.VMEM((CPR, CHUNK), jnp.int32),
        pltpu.VMEM((CHUNK, DHID), jnp.float32),
        pltpu.VMEM((CHUNK, DHID), jnp.float32),
        pltpu.VMEM_SHARED((ACC_ROWS, DHID), jnp.float32),
        pltpu.SemaphoreType.DMA,
        pltpu.SemaphoreType.DMA,
    ],
)
def _agg_kernel(vt_hbm, src_hbm, dst_hbm, zrows_hbm, out_hbm,
                src_v, dst_v, gb0, gb1, acc, sem0, sem1):
    c = lax.axis_index("c")
    s = lax.axis_index("s")
    pltpu.sync_copy(zrows_hbm, acc.at[pl.ds(s * 640, 640)])
    pltpu.sync_copy(src_hbm.at[c, s], src_v)
    pltpu.sync_copy(dst_hbm.at[c, s], dst_v)
    plsc.subcore_barrier()

    pltpu.async_copy(vt_hbm.at[src_v.at[0]], gb0, sem0)

    def body(i, carry):
        j = 2 * i
        pltpu.async_copy(vt_hbm.at[src_v.at[j + 1]], gb1, sem1)
        pltpu.make_async_copy(vt_hbm.at[src_v.at[0]], gb0, sem0).wait()
        pltpu.sync_copy(gb0, acc.at[dst_v.at[j]], add=True)
        pltpu.async_copy(vt_hbm.at[src_v.at[j + 2]], gb0, sem0)
        pltpu.make_async_copy(vt_hbm.at[src_v.at[0]], gb1, sem1).wait()
        pltpu.sync_copy(gb1, acc.at[dst_v.at[j + 1]], add=True)
        return carry

    lax.fori_loop(0, NPAIR, body, 0)
    pltpu.make_async_copy(vt_hbm.at[src_v.at[0]], gb0, sem0).wait()
    plsc.subcore_barrier()

    @pl.when(s < NTILES - 1)
    def _():
        pltpu.sync_copy(acc.at[pl.ds(s * 640, 640)],
                        out_hbm.at[pl.ds(c * NN + s * 640, 640)])

    @pl.when(s == NTILES - 1)
    def _():
        pltpu.sync_copy(acc.at[pl.ds(9600, 400)],
                        out_hbm.at[pl.ds(c * NN + 9600, 400)])


_BA = 2000


def _stage_a_body(x_ref, w_ref, deg_ref, o_ref):
    dinv = lax.rsqrt(deg_ref[...])
    xw = jnp.dot(x_ref[...], w_ref[...], preferred_element_type=jnp.float32)
    o_ref[...] = xw * dinv


def _stage_b_body(s_ref, v_ref, deg_ref, b_ref, o_ref):
    dinv = lax.rsqrt(deg_ref[...])
    z = dinv * (s_ref[...] + v_ref[...]) + b_ref[...]
    h = jnp.where(z > 0, z, jnp.exp(z) - 1.0)
    o_ref[...] = dinv * h


def _stage_c_body(s_ref, v_ref, deg_ref, w_ref, b_ref, o_ref):
    dinv = lax.rsqrt(deg_ref[...])
    agg = dinv * (s_ref[...] + v_ref[...])
    z = jnp.dot(agg, w_ref[...], preferred_element_type=jnp.float32) + b_ref[...]
    o_ref[...] = jnp.where(z > 0, z, jnp.exp(z) - 1.0)


def _stage_a(xs, ws, degs):
    return pl.pallas_call(
        _stage_a_body,
        grid=(2, NN // _BA),
        in_specs=[
            pl.BlockSpec((None, _BA, DIN), lambda g, i: (g, i, 0)),
            pl.BlockSpec((None, DIN, DHID), lambda g, i: (g, 0, 0)),
            pl.BlockSpec((None, _BA, 1), lambda g, i: (g, i, 0)),
        ],
        out_specs=pl.BlockSpec((None, _BA, DHID), lambda g, i: (g, i, 0)),
        out_shape=jax.ShapeDtypeStruct((2, NN, DHID), jnp.float32),
    )(xs, ws, degs)


def _stage_b(s1, vt1, degs, bs):
    return pl.pallas_call(
        _stage_b_body,
        grid=(2, NN // _BA),
        in_specs=[
            pl.BlockSpec((None, _BA, DHID), lambda g, i: (g, i, 0)),
            pl.BlockSpec((None, _BA, DHID), lambda g, i: (g, i, 0)),
            pl.BlockSpec((None, _BA, 1), lambda g, i: (g, i, 0)),
            pl.BlockSpec((None, 1, DHID), lambda g, i: (g, 0, 0)),
        ],
        out_specs=pl.BlockSpec((None, _BA, DHID), lambda g, i: (g, i, 0)),
        out_shape=jax.ShapeDtypeStruct((2, NN, DHID), jnp.float32),
    )(s1, vt1, degs, bs)


def _stage_c(s2, vt2, degs, ws, bs):
    return pl.pallas_call(
        _stage_c_body,
        grid=(2, NN // _BA),
        in_specs=[
            pl.BlockSpec((None, _BA, DHID), lambda g, i: (g, i, 0)),
            pl.BlockSpec((None, _BA, DHID), lambda g, i: (g, i, 0)),
            pl.BlockSpec((None, _BA, 1), lambda g, i: (g, i, 0)),
            pl.BlockSpec((None, DHID, DOUT), lambda g, i: (g, 0, 0)),
            pl.BlockSpec((None, 1, DOUT), lambda g, i: (g, 0, 0)),
        ],
        out_specs=pl.BlockSpec((None, _BA, DOUT), lambda g, i: (g, i, 0)),
        out_shape=jax.ShapeDtypeStruct((2, NN, DOUT), jnp.float32),
    )(s2, vt2, degs, ws, bs)


def kernel(x0, edge_index0, x1, edge_index1,
           W1_0, b1_0, W2_0, b2_0, W1_1, b1_1, W2_1, b2_1):
    xs = jnp.stack([x0, x1])
    w1 = jnp.stack([W1_0, W1_1])
    w2 = jnp.stack([W2_0, W2_1])
    b1 = jnp.stack([b1_0, b1_1])[:, None, :]
    b2 = jnp.stack([b2_0, b2_1])[:, None, :]

    pad = NTILES * CPT * CHUNK - EE

    def prep(ei, g):
        src = jnp.concatenate(
            [ei[0] + g * NN, jnp.full((pad,), g * NN, jnp.int32)])
        dst = jnp.concatenate([ei[1], jnp.full((pad,), NN, jnp.int32)])
        src = src.reshape(NTILES, CPT, CHUNK)
        dst = dst.reshape(NTILES, CPT, CHUNK)
        dummy_s = jnp.full((NTILES, CPR - CPT, CHUNK), g * NN, jnp.int32)
        dummy_d = jnp.full((NTILES, CPR - CPT, CHUNK), NN, jnp.int32)
        return (jnp.concatenate([src, dummy_s], axis=1),
                jnp.concatenate([dst, dummy_d], axis=1))

    s0, d0 = prep(edge_index0, 0)
    s1e, d1e = prep(edge_index1, 1)
    srcs = jnp.stack([s0, s1e])
    dsts = jnp.stack([d0, d1e])

    ones640 = jnp.ones((640, DEGW), jnp.float32)
    zrows = jnp.zeros((640, DHID), jnp.float32)

    deg = _deg_kernel(dsts, ones640)[:, :1].reshape(2, NN, 1)
    vt1 = _stage_a(xs, w1, deg)
    s1 = _agg_kernel(vt1.reshape(2 * NN, DHID), srcs, dsts, zrows)
    vt2 = _stage_b(s1.reshape(2, NN, DHID), vt1, deg, b1)
    s2 = _agg_kernel(vt2.reshape(2 * NN, DHID), srcs, dsts, zrows)
    out = _stage_c(s2.reshape(2, NN, DHID), vt2, deg, w2, b2)
    return out.reshape(2 * NN, DOUT)
